# baseline (speedup 1.0000x reference)
import functools

import jax
import jax.numpy as jnp
from jax import lax
from jax.experimental import pallas as pl
from jax.experimental.pallas import tpu as pltpu
from jax.experimental.pallas import tpu_sc as plsc

SEQ = 4096
D = 128
NC = 2
NS = 16
NW = NC * NS
BPW = SEQ // NW
L = 16
NCH = D // L


def _tree_sum(vs):
    while len(vs) > 1:
        vs = [a + b for a, b in zip(vs[::2], vs[1::2])]
    return vs[0]


def _rsqrt_newton(v):
    i = lax.bitcast_convert_type(v, jnp.int32)
    i = jnp.int32(0x5F3759DF) - lax.shift_right_logical(i, 1)
    y = lax.bitcast_convert_type(i, jnp.float32)
    y = y * (1.5 - (0.5 * v) * y * y)
    return y


def _lane_total(x):
    iota = lax.iota(jnp.int32, L)
    for k in (1, 2, 4, 8):
        x = x + x.at[iota ^ k].get(mode="promise_in_bounds")
    return x


def _sc_embed(idx_hbm, table_hbm, pos_hbm, w_hbm, b_hbm, out_hbm,
              idx_v, rows_v, pos_v, out_v, w_v, b_v,
              sem_p, sem_g, sem_o):
    wid = lax.axis_index("s") * NC + lax.axis_index("c")
    base = wid * BPW
    half = BPW // 2
    pltpu.sync_copy(idx_hbm.at[pl.ds(base, BPW)], idx_v)
    pltpu.async_copy(table_hbm.at[idx_v.at[pl.ds(0, half)]],
                     rows_v.at[pl.ds(0, half)], sem_g.at[0])
    pltpu.async_copy(table_hbm.at[idx_v.at[pl.ds(half, half)]],
                     rows_v.at[pl.ds(half, half)], sem_g.at[1])
    pos = pltpu.async_copy(pos_hbm.at[pl.ds(base, BPW)], pos_v, sem_p)
    pltpu.sync_copy(w_hbm, w_v)
    pltpu.sync_copy(b_hbm, b_v)
    pos.wait()

    w = [w_v[pl.ds(c * L, L)] for c in range(NCH)]
    b = [b_v[pl.ds(c * L, L)] for c in range(NCH)]

    def blk(k, carry):
        lo = k * half
        pltpu.make_async_copy(table_hbm.at[idx_v.at[pl.ds(lo, half)]],
                              rows_v.at[pl.ds(lo, half)], sem_g.at[k]).wait()

        @plsc.parallel_loop(lo, lo + half, unroll=1)
        def row(r):
            xs = [rows_v[r, pl.ds(c * L, L)] + pos_v[r, pl.ds(c * L, L)]
                  for c in range(NCH)]
            s = _lane_total(_tree_sum(xs))
            q = _lane_total(_tree_sum([x * x for x in xs]))
            mean_v = s * (1.0 / D)
            var_v = q * (1.0 / D) - mean_v * mean_v
            inv_v = _rsqrt_newton(var_v + 1e-5)
            nm = -mean_v
            for c in range(NCH):
                iw = inv_v * w[c]
                out_v[r, pl.ds(c * L, L)] = (xs[c] + nm) * iw + b[c]

        return carry

    lax.fori_loop(0, 2, blk, 0)
    pltpu.sync_copy(out_v, out_hbm.at[pl.ds(base, BPW)])


def kernel(token_ids, token_table, pos_table, ln_weight, ln_bias):
    mesh = plsc.VectorSubcoreMesh(core_axis_name="c", subcore_axis_name="s")
    embed = functools.partial(
        pl.kernel,
        mesh=mesh,
        out_type=jax.ShapeDtypeStruct((SEQ, D), jnp.float32),
        scratch_types=[
            pltpu.VMEM((BPW,), jnp.int32),
            pltpu.VMEM((BPW, D), jnp.float32),
            pltpu.VMEM((BPW, D), jnp.float32),
            pltpu.VMEM((BPW, D), jnp.float32),
            pltpu.VMEM((D,), jnp.float32),
            pltpu.VMEM((D,), jnp.float32),
            pltpu.SemaphoreType.DMA,
            pltpu.SemaphoreType.DMA((2,)),
            pltpu.SemaphoreType.DMA,
        ],
    )(_sc_embed)
    return embed(token_ids.astype(jnp.int32), token_table, pos_table,
                 ln_weight, ln_bias)


# speedup vs baseline: 1.0048x; 1.0048x over previous
import functools

import jax
import jax.numpy as jnp
from jax import lax
from jax.experimental import pallas as pl
from jax.experimental.pallas import tpu as pltpu
from jax.experimental.pallas import tpu_sc as plsc

SEQ = 4096
D = 128
NC = 2
NS = 16
NW = NC * NS
BPW = SEQ // NW
L = 16
NCH = D // L


def _tree_sum(vs):
    while len(vs) > 1:
        vs = [a + b for a, b in zip(vs[::2], vs[1::2])]
    return vs[0]


def _rsqrt_newton(v):
    i = lax.bitcast_convert_type(v, jnp.int32)
    i = jnp.int32(0x5F3759DF) - lax.shift_right_logical(i, 1)
    y = lax.bitcast_convert_type(i, jnp.float32)
    y = y * (1.5 - (0.5 * v) * y * y)
    return y


def _lane_total(x):
    iota = lax.iota(jnp.int32, L)
    for k in (1, 2, 4, 8):
        x = x + x.at[iota ^ k].get(mode="promise_in_bounds")
    return x


def _sc_embed(idx_hbm, table_hbm, pos_hbm, w_hbm, b_hbm, out_hbm,
              idx_v, rows_v, pos_v, out_v, w_v, b_v,
              sem_p, sem_g, sem_o):
    wid = lax.axis_index("s") * NC + lax.axis_index("c")
    base = wid * BPW
    pltpu.sync_copy(idx_hbm.at[pl.ds(base, BPW)], idx_v)
    gather = pltpu.async_copy(table_hbm.at[idx_v], rows_v, sem_g)
    pos = pltpu.async_copy(pos_hbm.at[pl.ds(base, BPW)], pos_v, sem_p)
    pltpu.sync_copy(w_hbm, w_v)
    pltpu.sync_copy(b_hbm, b_v)
    pos.wait()
    gather.wait()

    w = [w_v[pl.ds(c * L, L)] for c in range(NCH)]
    b = [b_v[pl.ds(c * L, L)] for c in range(NCH)]

    @plsc.parallel_loop(0, BPW, unroll=1)
    def row(r):
        xs = [rows_v[r, pl.ds(c * L, L)] + pos_v[r, pl.ds(c * L, L)]
              for c in range(NCH)]
        s = _lane_total(_tree_sum(xs))
        q = _lane_total(_tree_sum([x * x for x in xs]))
        mean_v = s * (1.0 / D)
        var_v = q * (1.0 / D) - mean_v * mean_v
        inv_v = _rsqrt_newton(var_v + 1e-5)
        nm = -mean_v
        for c in range(NCH):
            iw = inv_v * w[c]
            out_v[r, pl.ds(c * L, L)] = (xs[c] + nm) * iw + b[c]

    pltpu.sync_copy(out_v, out_hbm.at[pl.ds(base, BPW)])


def kernel(token_ids, token_table, pos_table, ln_weight, ln_bias):
    mesh = plsc.VectorSubcoreMesh(core_axis_name="c", subcore_axis_name="s")
    embed = functools.partial(
        pl.kernel,
        mesh=mesh,
        out_type=jax.ShapeDtypeStruct((SEQ, D), jnp.float32),
        scratch_types=[
            pltpu.VMEM((BPW,), jnp.int32),
            pltpu.VMEM((BPW, D), jnp.float32),
            pltpu.VMEM((BPW, D), jnp.float32),
            pltpu.VMEM((BPW, D), jnp.float32),
            pltpu.VMEM((D,), jnp.float32),
            pltpu.VMEM((D,), jnp.float32),
            pltpu.SemaphoreType.DMA,
            pltpu.SemaphoreType.DMA,
            pltpu.SemaphoreType.DMA,
        ],
    )(_sc_embed)
    return embed(token_ids.astype(jnp.int32), token_table, pos_table,
                 ln_weight, ln_bias)


# speedup vs baseline: 1.0065x; 1.0017x over previous
import functools

import jax
import jax.numpy as jnp
from jax import lax
from jax.experimental import pallas as pl
from jax.experimental.pallas import tpu as pltpu
from jax.experimental.pallas import tpu_sc as plsc

SEQ = 4096
D = 128
NC = 2
NS = 16
NW = NC * NS
BPW = SEQ // NW
L = 16
NCH = D // L


def _tree_sum(vs):
    while len(vs) > 1:
        vs = [a + b for a, b in zip(vs[::2], vs[1::2])]
    return vs[0]


def _rsqrt_newton(v):
    i = lax.bitcast_convert_type(v, jnp.int32)
    i = jnp.int32(0x5F3759DF) - lax.shift_right_logical(i, 1)
    y = lax.bitcast_convert_type(i, jnp.float32)
    y = y * (1.5 - (0.5 * v) * y * y)
    return y


def _lane_total(x, perms):
    for p in perms:
        x = x + x.at[p].get(mode="promise_in_bounds")
    return x


def _sc_embed(idx_hbm, table_hbm, pos_hbm, w_hbm, b_hbm, out_hbm,
              idx_v, rows_v, pos_v, out_v, w_v, b_v,
              sem_p, sem_g, sem_o):
    wid = lax.axis_index("s") * NC + lax.axis_index("c")
    base = wid * BPW
    pltpu.sync_copy(idx_hbm.at[pl.ds(base, BPW)], idx_v)
    gather = pltpu.async_copy(table_hbm.at[idx_v], rows_v, sem_g)
    pos = pltpu.async_copy(pos_hbm.at[pl.ds(base, BPW)], pos_v, sem_p)
    pltpu.sync_copy(w_hbm, w_v)
    pltpu.sync_copy(b_hbm, b_v)
    pos.wait()
    gather.wait()

    w = [w_v[pl.ds(c * L, L)] for c in range(NCH)]
    b = [b_v[pl.ds(c * L, L)] for c in range(NCH)]
    iota = lax.iota(jnp.int32, L)
    perms = [iota ^ k for k in (1, 2, 4, 8)]

    @plsc.parallel_loop(0, BPW, unroll=1)
    def row(r):
        xs = [rows_v[r, pl.ds(c * L, L)] + pos_v[r, pl.ds(c * L, L)]
              for c in range(NCH)]
        s = _lane_total(_tree_sum(xs), perms)
        q = _lane_total(_tree_sum([x * x for x in xs]), perms)
        mean_v = s * (1.0 / D)
        var_v = q * (1.0 / D) - mean_v * mean_v
        inv_v = _rsqrt_newton(var_v + 1e-5)
        nm = -mean_v
        for c in range(NCH):
            iw = inv_v * w[c]
            out_v[r, pl.ds(c * L, L)] = (xs[c] + nm) * iw + b[c]

    pltpu.sync_copy(out_v, out_hbm.at[pl.ds(base, BPW)])


def kernel(token_ids, token_table, pos_table, ln_weight, ln_bias):
    mesh = plsc.VectorSubcoreMesh(core_axis_name="c", subcore_axis_name="s")
    embed = functools.partial(
        pl.kernel,
        mesh=mesh,
        out_type=jax.ShapeDtypeStruct((SEQ, D), jnp.float32),
        scratch_types=[
            pltpu.VMEM((BPW,), jnp.int32),
            pltpu.VMEM((BPW, D), jnp.float32),
            pltpu.VMEM((BPW, D), jnp.float32),
            pltpu.VMEM((BPW, D), jnp.float32),
            pltpu.VMEM((D,), jnp.float32),
            pltpu.VMEM((D,), jnp.float32),
            pltpu.SemaphoreType.DMA,
            pltpu.SemaphoreType.DMA,
            pltpu.SemaphoreType.DMA,
        ],
    )(_sc_embed)
    return embed(token_ids.astype(jnp.int32), token_table, pos_table,
                 ln_weight, ln_bias)


# speedup vs baseline: 1.0072x; 1.0007x over previous
import functools

import jax
import jax.numpy as jnp
from jax import lax
from jax.experimental import pallas as pl
from jax.experimental.pallas import tpu as pltpu
from jax.experimental.pallas import tpu_sc as plsc

SEQ = 4096
D = 128
NC = 2
NS = 16
NW = NC * NS
BPW = SEQ // NW
L = 16
NCH = D // L


def _tree_sum(vs):
    while len(vs) > 1:
        vs = [a + b for a, b in zip(vs[::2], vs[1::2])]
    return vs[0]


def _rsqrt_newton(v):
    i = lax.bitcast_convert_type(v, jnp.int32)
    i = jnp.int32(0x5F3759DF) - lax.shift_right_logical(i, 1)
    y = lax.bitcast_convert_type(i, jnp.float32)
    y = y * (1.5 - (0.5 * v) * y * y)
    return y


def _lane_total(x, perms):
    for p in perms:
        x = x + x.at[p].get(mode="promise_in_bounds")
    return x


def _sc_embed(idx_hbm, table_hbm, pos_hbm, w_hbm, b_hbm, out_hbm,
              idx_v, rows_v, pos_v, out_v, w_v, b_v,
              sem_p, sem_g):
    wid = lax.axis_index("s") * NC + lax.axis_index("c")
    base = wid * BPW
    pltpu.sync_copy(idx_hbm.at[pl.ds(base, BPW)], idx_v)
    gather = pltpu.async_copy(table_hbm.at[idx_v], rows_v, sem_g)
    pos = pltpu.async_copy(pos_hbm.at[pl.ds(base, BPW)], pos_v, sem_p)
    pltpu.sync_copy(w_hbm, w_v)
    pltpu.sync_copy(b_hbm, b_v)
    pos.wait()
    gather.wait()

    w = [w_v[pl.ds(c * L, L)] for c in range(NCH)]
    b = [b_v[pl.ds(c * L, L)] for c in range(NCH)]
    iota = lax.iota(jnp.int32, L)
    perms = [iota ^ k for k in (1, 2, 4, 8)]

    @plsc.parallel_loop(0, BPW, unroll=1)
    def row(r):
        xs = [rows_v[r, pl.ds(c * L, L)] + pos_v[r, pl.ds(c * L, L)]
              for c in range(NCH)]
        s = _lane_total(_tree_sum(xs), perms)
        q = _lane_total(_tree_sum([x * x for x in xs]), perms)
        mean_v = s * (1.0 / D)
        var_v = q * (1.0 / D) - mean_v * mean_v
        inv_v = _rsqrt_newton(var_v + 1e-5)
        nm = -mean_v
        for c in range(NCH):
            iw = inv_v * w[c]
            out_v[r, pl.ds(c * L, L)] = (xs[c] + nm) * iw + b[c]

    pltpu.sync_copy(out_v, out_hbm.at[pl.ds(base, BPW)])


def kernel(token_ids, token_table, pos_table, ln_weight, ln_bias):
    mesh = plsc.VectorSubcoreMesh(core_axis_name="c", subcore_axis_name="s")
    embed = functools.partial(
        pl.kernel,
        mesh=mesh,
        out_type=jax.ShapeDtypeStruct((SEQ, D), jnp.float32),
        scratch_types=[
            pltpu.VMEM((BPW,), jnp.int32),
            pltpu.VMEM((BPW, D), jnp.float32),
            pltpu.VMEM((BPW, D), jnp.float32),
            pltpu.VMEM((BPW, D), jnp.float32),
            pltpu.VMEM((D,), jnp.float32),
            pltpu.VMEM((D,), jnp.float32),
            pltpu.SemaphoreType.DMA,
            pltpu.SemaphoreType.DMA,
        ],
    )(_sc_embed)
    return embed(token_ids.astype(jnp.int32), token_table, pos_table,
                 ln_weight, ln_bias)
